# baseline (speedup 1.0000x reference)
import functools

import jax
import jax.numpy as jnp
from jax import lax
from jax.experimental import pallas as pl
from jax.experimental.pallas import tpu as pltpu
from jax.experimental.pallas import tpu_sc as plsc

_N = 2_000_000
_NC = 2
_NS = 16
_NW = _NC * _NS
_L = 16
_ROW = 128
_CHUNK = 61 * _ROW
_NCHUNK = 8
_PER_TILE = _CHUNK * _NCHUNK
_TAIL_ROW0 = _NW * _PER_TILE // _ROW
_NTAIL = _N // _ROW - _TAIL_ROW0
_TBL = 119
_TBL_PAD = 128

_NORM = 0.5

_mesh = plsc.VectorSubcoreMesh(core_axis_name="c", subcore_axis_name="s")


@functools.partial(
    pl.kernel,
    mesh=_mesh,
    compiler_params=pltpu.CompilerParams(needs_layout_passes=False),
    out_type=jax.ShapeDtypeStruct((_N,), jnp.float32),
    scratch_types=[
        pltpu.VMEM((_TBL_PAD,), jnp.float32),
        pltpu.VMEM((_CHUNK,), jnp.int32),
        pltpu.VMEM((_CHUNK,), jnp.int32),
        pltpu.VMEM((_CHUNK,), jnp.float32),
        pltpu.VMEM((_CHUNK,), jnp.float32),
        pltpu.VMEM((_TBL_PAD,), jnp.float32),
        pltpu.VMEM((_ROW,), jnp.int32),
        pltpu.VMEM((_ROW,), jnp.float32),
        pltpu.SemaphoreType.DMA,
        pltpu.SemaphoreType.DMA,
        pltpu.SemaphoreType.DMA,
        pltpu.SemaphoreType.DMA,
    ],
)
def _fixed_charges_sc(idx_hbm, tbl_hbm, out_hbm, tbl_v, idx_v0, idx_v1,
                      out_v0, out_v1, tbl_raw, tidx_v, tout_v,
                      sem_i0, sem_i1, sem_o0, sem_o1):
    wid = lax.axis_index("s") * _NC + lax.axis_index("c")
    base = wid * _PER_TILE

    idx_bufs = (idx_v0, idx_v1)
    out_bufs = (out_v0, out_v1)
    sems_in = (sem_i0, sem_i1)
    sems_out = (sem_o0, sem_o1)

    for b in range(2):
        pltpu.async_copy(
            idx_hbm.at[pl.ds(base + b * _CHUNK, _CHUNK)], idx_bufs[b],
            sems_in[b],
        )

    pltpu.sync_copy(tbl_hbm, tbl_raw.at[pl.ds(0, _TBL)])
    for s in list(range(0, _TBL - _L, _L)) + [_TBL - _L]:
        sl = pl.ds(s, _L)
        tbl_v[sl] = tbl_raw[sl] * jnp.float32(_NORM)

    @pl.when(wid < _NTAIL)
    def _tail():
        off = (_TAIL_ROW0 + wid) * _ROW
        pltpu.sync_copy(idx_hbm.at[pl.ds(off, _ROW)], tidx_v)
        for i in range(_ROW // _L):
            sl = pl.ds(i * _L, _L)
            tout_v[sl] = plsc.load_gather(tbl_v, [tidx_v[sl]])
        pltpu.sync_copy(tout_v, out_hbm.at[pl.ds(off, _ROW)])

    @pl.loop(0, _NCHUNK, step=2)
    def _ring(g0):
        for b in range(2):
            gg = g0 + b
            off = base + gg * _CHUNK
            ib = idx_bufs[b]
            ob = out_bufs[b]

            pltpu.make_async_copy(
                idx_hbm.at[pl.ds(off, _CHUNK)], ib, sems_in[b]
            ).wait()

            @pl.when(gg >= 2)
            def _wait_prev_out():
                pltpu.make_async_copy(
                    ob, out_hbm.at[pl.ds(off, _CHUNK)], sems_out[b]
                ).wait()

            @plsc.parallel_loop(0, _CHUNK, step=_L, unroll=4)
            def _gather(i):
                ob[pl.ds(i, _L)] = plsc.load_gather(tbl_v, [ib[pl.ds(i, _L)]])

            pltpu.async_copy(ob, out_hbm.at[pl.ds(off, _CHUNK)], sems_out[b])

            @pl.when(gg + 2 < _NCHUNK)
            def _next_in():
                off2 = base + (gg + 2) * _CHUNK
                pltpu.async_copy(
                    idx_hbm.at[pl.ds(off2, _CHUNK)], ib, sems_in[b]
                )

    for b in range(2):
        pltpu.make_async_copy(
            out_bufs[b], out_hbm.at[pl.ds(base, _CHUNK)], sems_out[b]
        ).wait()


def kernel(atomic_numbers, charge_table):
    return _fixed_charges_sc(
        atomic_numbers.astype(jnp.int32), charge_table.astype(jnp.float32)
    )


# speedup vs baseline: 1.0155x; 1.0155x over previous
import functools

import jax
import jax.numpy as jnp
from jax import lax
from jax.experimental import pallas as pl
from jax.experimental.pallas import tpu as pltpu
from jax.experimental.pallas import tpu_sc as plsc

_N = 2_000_000
_NC = 2
_NS = 16
_NW = _NC * _NS
_L = 16
_ROW = 128
_CHUNK = 61 * _ROW
_NCHUNK = 8
_NBUF = 4
_PER_TILE = _CHUNK * _NCHUNK
_TAIL_ROW0 = _NW * _PER_TILE // _ROW
_NTAIL = _N // _ROW - _TAIL_ROW0
_TBL = 119
_TBL_PAD = 128

_NORM = 0.5

_mesh = plsc.VectorSubcoreMesh(core_axis_name="c", subcore_axis_name="s")


@functools.partial(
    pl.kernel,
    mesh=_mesh,
    compiler_params=pltpu.CompilerParams(needs_layout_passes=False),
    out_type=jax.ShapeDtypeStruct((_N,), jnp.float32),
    scratch_types=(
        [pltpu.VMEM((_TBL_PAD,), jnp.float32)]
        + [pltpu.VMEM((_CHUNK,), jnp.int32) for _ in range(_NBUF)]
        + [pltpu.VMEM((_CHUNK,), jnp.float32) for _ in range(_NBUF)]
        + [
            pltpu.VMEM((_TBL_PAD,), jnp.float32),
            pltpu.VMEM((_ROW,), jnp.int32),
            pltpu.VMEM((_ROW,), jnp.float32),
        ]
        + [pltpu.SemaphoreType.DMA for _ in range(2 * _NBUF)]
    ),
)
def _fixed_charges_sc(idx_hbm, tbl_hbm, out_hbm, tbl_v,
                      idx_v0, idx_v1, idx_v2, idx_v3,
                      out_v0, out_v1, out_v2, out_v3,
                      tbl_raw, tidx_v, tout_v,
                      sem_i0, sem_i1, sem_i2, sem_i3,
                      sem_o0, sem_o1, sem_o2, sem_o3):
    wid = lax.axis_index("s") * _NC + lax.axis_index("c")
    base = wid * _PER_TILE

    idx_bufs = (idx_v0, idx_v1, idx_v2, idx_v3)
    out_bufs = (out_v0, out_v1, out_v2, out_v3)
    sems_in = (sem_i0, sem_i1, sem_i2, sem_i3)
    sems_out = (sem_o0, sem_o1, sem_o2, sem_o3)

    for b in range(_NBUF):
        pltpu.async_copy(
            idx_hbm.at[pl.ds(base + b * _CHUNK, _CHUNK)], idx_bufs[b],
            sems_in[b],
        )

    pltpu.sync_copy(tbl_hbm, tbl_raw.at[pl.ds(0, _TBL)])
    for s in list(range(0, _TBL - _L, _L)) + [_TBL - _L]:
        sl = pl.ds(s, _L)
        tbl_v[sl] = tbl_raw[sl] * jnp.float32(_NORM)

    @pl.when(wid < _NTAIL)
    def _tail():
        off = (_TAIL_ROW0 + wid) * _ROW
        pltpu.sync_copy(idx_hbm.at[pl.ds(off, _ROW)], tidx_v)
        for i in range(_ROW // _L):
            sl = pl.ds(i * _L, _L)
            tout_v[sl] = plsc.load_gather(tbl_v, [tidx_v[sl]])
        pltpu.sync_copy(tout_v, out_hbm.at[pl.ds(off, _ROW)])

    @pl.loop(0, _NCHUNK, step=_NBUF)
    def _ring(g0):
        for b in range(_NBUF):
            gg = g0 + b
            off = base + gg * _CHUNK
            ib = idx_bufs[b]
            ob = out_bufs[b]

            pltpu.make_async_copy(
                idx_hbm.at[pl.ds(off, _CHUNK)], ib, sems_in[b]
            ).wait()

            @pl.when(gg >= _NBUF)
            def _wait_prev_out():
                pltpu.make_async_copy(
                    ob, out_hbm.at[pl.ds(off, _CHUNK)], sems_out[b]
                ).wait()

            @plsc.parallel_loop(0, _CHUNK, step=_L, unroll=4)
            def _gather(i):
                ob[pl.ds(i, _L)] = plsc.load_gather(tbl_v, [ib[pl.ds(i, _L)]])

            pltpu.async_copy(ob, out_hbm.at[pl.ds(off, _CHUNK)], sems_out[b])

            @pl.when(gg + _NBUF < _NCHUNK)
            def _next_in():
                off2 = base + (gg + _NBUF) * _CHUNK
                pltpu.async_copy(
                    idx_hbm.at[pl.ds(off2, _CHUNK)], ib, sems_in[b]
                )

    for b in range(_NBUF):
        pltpu.make_async_copy(
            out_bufs[b], out_hbm.at[pl.ds(base, _CHUNK)], sems_out[b]
        ).wait()


def kernel(atomic_numbers, charge_table):
    return _fixed_charges_sc(
        atomic_numbers.astype(jnp.int32), charge_table.astype(jnp.float32)
    )
